# baseline (speedup 1.0000x reference)
import jax
import jax.numpy as jnp
from jax.experimental import pallas as pl
from jax.experimental.pallas import tpu as pltpu

_N = 10000
_D = 128
_OUT = 128
_BM = 200
_K = 2


def _sage_fused_kernel(*refs):
    a_refs = refs[:_K]
    f_all_ref = refs[_K]
    f_blk_refs = refs[_K + 1:2 * _K + 1]
    wt_ref = refs[2 * _K + 1]
    out_ref = refs[2 * _K + 2]
    f_all = f_all_ref[...]
    w1 = wt_ref[:_D, :]
    w2 = wt_ref[_D:, :]
    for j in range(_K):
        a = a_refs[j][...]
        neigh = jnp.dot(a, f_all,
                        preferred_element_type=jnp.float32)
        rowsum = jnp.sum(a, axis=1, keepdims=True)
        neigh = neigh / (rowsum + 1.0)
        out = jnp.dot(f_blk_refs[j][...], w1,
                      preferred_element_type=jnp.float32)
        out = out + jnp.dot(neigh, w2,
                            preferred_element_type=jnp.float32)
        out_ref[j * _BM:(j + 1) * _BM, :] = out


def _adj_spec(j):
    return pl.BlockSpec((_BM, _N), lambda i, j=j: (_K * i + j, 0))


def _fblk_spec(j):
    return pl.BlockSpec((_BM, _D), lambda i, j=j: (_K * i + j, 0))


def kernel(adj, features, W):
    wt = W.T
    grid = _N // (_K * _BM)
    in_specs = (
        [_adj_spec(j) for j in range(_K)]
        + [pl.BlockSpec((_N, _D), lambda i: (0, 0))]
        + [_fblk_spec(j) for j in range(_K)]
        + [pl.BlockSpec((2 * _D, _OUT), lambda i: (0, 0))]
    )
    return pl.pallas_call(
        _sage_fused_kernel,
        grid=(grid,),
        in_specs=in_specs,
        out_specs=pl.BlockSpec((_K * _BM, _OUT), lambda i: (i, 0)),
        out_shape=jax.ShapeDtypeStruct((_N, _OUT), jnp.float32),
    )(*([adj] * _K), features, *([features] * _K), wt)


# speedup vs baseline: 1.0360x; 1.0360x over previous
import jax
import jax.numpy as jnp
from jax.experimental import pallas as pl
from jax.experimental.pallas import tpu as pltpu

_N = 10000
_D = 128
_OUT = 128
_BM = 200
_R = 4
_STEPS = _N // _BM


def _sage_kernel(adj_hbm, f_all_ref, wt_ref, out_ref, buf, sems):
    def _copy(step, slot):
        return pltpu.make_async_copy(
            adj_hbm.at[pl.ds(step * _BM, _BM), :],
            buf.at[slot],
            sems.at[slot],
        )

    for s in range(_R - 1):
        _copy(s, s).start()

    f_all = f_all_ref[...]
    w1 = wt_ref[:_D, :]
    w2 = wt_ref[_D:, :]

    for i in range(_STEPS):
        slot = i % _R
        _copy(i, slot).wait()
        nxt = i + _R - 1
        if nxt < _STEPS:
            _copy(nxt, nxt % _R).start()
        a = buf[slot]
        neigh = jnp.dot(a, f_all,
                        preferred_element_type=jnp.float32)
        rowsum = jnp.sum(a, axis=1, keepdims=True)
        neigh = neigh / (rowsum + 1.0)
        out = jnp.dot(f_all_ref[pl.ds(i * _BM, _BM), :], w1,
                      preferred_element_type=jnp.float32)
        out = out + jnp.dot(neigh, w2,
                            preferred_element_type=jnp.float32)
        out_ref[pl.ds(i * _BM, _BM), :] = out


def kernel(adj, features, W):
    wt = W.T
    return pl.pallas_call(
        _sage_kernel,
        in_specs=[
            pl.BlockSpec(memory_space=pltpu.HBM),
            pl.BlockSpec(memory_space=pltpu.VMEM),
            pl.BlockSpec(memory_space=pltpu.VMEM),
        ],
        out_specs=pl.BlockSpec(memory_space=pltpu.VMEM),
        out_shape=jax.ShapeDtypeStruct((_N, _OUT), jnp.float32),
        scratch_shapes=[
            pltpu.VMEM((_R, _BM, _N), jnp.float32),
            pltpu.SemaphoreType.DMA((_R,)),
        ],
    )(adj, features, wt)


# speedup vs baseline: 1.0627x; 1.0258x over previous
import jax
import jax.numpy as jnp
from jax.experimental import pallas as pl
from jax.experimental.pallas import tpu as pltpu

_N = 10000
_D = 128
_OUT = 128
_BM = 400
_R = 3
_STEPS = _N // _BM


def _sage_kernel(adj_hbm, f_all_ref, wt_ref, out_hbm, buf, sems,
                 out_stage, out_sems):
    def _copy(step, slot):
        return pltpu.make_async_copy(
            adj_hbm.at[pl.ds(step * _BM, _BM), :],
            buf.at[slot],
            sems.at[slot],
        )

    def _out_copy(step, slot):
        return pltpu.make_async_copy(
            out_stage.at[slot],
            out_hbm.at[pl.ds(step * _BM, _BM), :],
            out_sems.at[slot],
        )

    for s in range(_R - 1):
        _copy(s, s).start()

    f_all = f_all_ref[...]
    w1 = wt_ref[:_D, :]
    w2 = wt_ref[_D:, :]

    for i in range(_STEPS):
        slot = i % _R
        _copy(i, slot).wait()
        nxt = i + _R - 1
        if nxt < _STEPS:
            _copy(nxt, nxt % _R).start()
        a = buf[slot]
        neigh = jnp.dot(a, f_all,
                        preferred_element_type=jnp.float32)
        rowsum = jnp.sum(a, axis=1, keepdims=True)
        neigh = neigh / (rowsum + 1.0)
        out = jnp.dot(f_all_ref[pl.ds(i * _BM, _BM), :], w1,
                      preferred_element_type=jnp.float32)
        out = out + jnp.dot(neigh, w2,
                            preferred_element_type=jnp.float32)
        oslot = i % 2
        if i >= 2:
            _out_copy(i - 2, oslot).wait()
        out_stage[oslot] = out
        _out_copy(i, oslot).start()

    for i in range(_STEPS - 2, _STEPS):
        _out_copy(i, i % 2).wait()


def kernel(adj, features, W):
    wt = W.T
    return pl.pallas_call(
        _sage_kernel,
        in_specs=[
            pl.BlockSpec(memory_space=pltpu.HBM),
            pl.BlockSpec(memory_space=pltpu.VMEM),
            pl.BlockSpec(memory_space=pltpu.VMEM),
        ],
        out_specs=pl.BlockSpec(memory_space=pltpu.HBM),
        out_shape=jax.ShapeDtypeStruct((_N, _OUT), jnp.float32),
        scratch_shapes=[
            pltpu.VMEM((_R, _BM, _N), jnp.float32),
            pltpu.SemaphoreType.DMA((_R,)),
            pltpu.VMEM((2, _BM, _OUT), jnp.float32),
            pltpu.SemaphoreType.DMA((2,)),
        ],
    )(adj, features, wt)
